# baseline (speedup 1.0000x reference)
import functools

import jax
import jax.numpy as jnp
from jax.experimental import pallas as pl
from jax.experimental.pallas import tpu as pltpu

_CODE_NUM = 8
_D = 3
_IDX = 1


def _interp_kernel(coords_ref, code_ref, out_ref, *, cn, c, tp):
    scaled = (coords_ref[...] + 1.0) * ((cn - 1) / 2.0)
    grid_i = jax.lax.broadcasted_iota(jnp.int32, (cn, tp), 0).astype(jnp.float32)

    def hat(j):
        return jnp.maximum(0.0, 1.0 - jnp.abs(grid_i - scaled[j:j + 1, :]))

    h0 = hat(0)
    h1 = hat(1)
    h2 = hat(2)

    w_low = (h1[:, None, :] * h0[None, :, :]).reshape(cn * cn, tp)

    h2b = h2.astype(jnp.bfloat16)
    wlb = w_low.astype(jnp.bfloat16)
    w3 = (h2b[:, None, :] * wlb[None, :, :]).reshape(cn ** 3, tp)

    out_ref[0, :, :] = jnp.dot(code_ref[...], w3,
                               preferred_element_type=jnp.float32)


@jax.jit
def kernel(coords, shape_code):
    b, p, d = coords.shape
    c = shape_code.shape[0]
    cn = _CODE_NUM
    nblk = cn ** d

    code_r = jax.lax.slice_in_dim(
        shape_code, _IDX * nblk, (_IDX + 1) * nblk, axis=1).astype(jnp.bfloat16)

    npts = b * p
    tp = 16384
    if p % tp != 0:
        tp = 1024 if p % 1024 == 0 else p
    tiles_per_batch = p // tp
    kernel_fn = functools.partial(_interp_kernel, cn=cn, c=c, tp=tp)

    coords_t = jnp.pad(coords.reshape(npts, d).T.astype(jnp.float32),
                       ((0, 8 - d), (0, 0)))

    out = pl.pallas_call(
        kernel_fn,
        out_shape=jax.ShapeDtypeStruct((b, c, p), jnp.float32),
        grid=(b * tiles_per_batch,),
        in_specs=[
            pl.BlockSpec((8, tp), lambda i: (0, i)),
            pl.BlockSpec((c, cn ** 3), lambda i: (0, 0)),
        ],
        out_specs=pl.BlockSpec(
            (1, c, tp),
            lambda i, _t=tiles_per_batch: (i // _t, 0, i % _t)),
        compiler_params=pltpu.CompilerParams(
            dimension_semantics=("parallel",),
            vmem_limit_bytes=64 * 1024 * 1024,
        ),
    )(coords_t, code_r)

    return out.transpose(0, 2, 1)


# speedup vs baseline: 1.3059x; 1.3059x over previous
import functools

import jax
import jax.numpy as jnp
from jax.experimental import pallas as pl
from jax.experimental.pallas import tpu as pltpu

_CODE_NUM = 8
_D = 3
_IDX = 1


def _interp_kernel(coords_ref, code_ref, out_ref, *, cn, c, tp):
    scaled = (coords_ref[...] + 1.0) * ((cn - 1) / 2.0)
    grid_i = jax.lax.broadcasted_iota(jnp.int32, (cn, tp), 0).astype(jnp.float32)

    def hat(j):
        return jnp.maximum(0.0, 1.0 - jnp.abs(grid_i - scaled[j:j + 1, :]))

    h0 = hat(0)
    h1 = hat(1)
    h2 = hat(2)

    w_low = (h1[:, None, :] * h0[None, :, :]).reshape(cn * cn, tp)

    a = jnp.dot(code_ref[...], w_low,
                preferred_element_type=jnp.float32)

    acc = a[0:c, :] * h2[0:1, :]
    for i in range(1, cn):
        acc = acc + a[i * c:(i + 1) * c, :] * h2[i:i + 1, :]

    out_ref[0, :, :] = acc


@jax.jit
def kernel(coords, shape_code):
    b, p, d = coords.shape
    c = shape_code.shape[0]
    cn = _CODE_NUM
    nblk = cn ** d

    code = jax.lax.slice_in_dim(shape_code, _IDX * nblk, (_IDX + 1) * nblk, axis=1)
    code_r = (code.astype(jnp.float32)
              .reshape(c, cn, cn * cn)
              .transpose(1, 0, 2)
              .reshape(cn * c, cn * cn))

    npts = b * p
    tp = 16384
    if p % tp != 0:
        tp = 1024 if p % 1024 == 0 else p
    tiles_per_batch = p // tp
    kernel_fn = functools.partial(_interp_kernel, cn=cn, c=c, tp=tp)

    coords_t = coords.reshape(npts, d).T.astype(jnp.float32)

    out = pl.pallas_call(
        kernel_fn,
        out_shape=jax.ShapeDtypeStruct((b, c, p), jnp.float32),
        grid=(b * tiles_per_batch,),
        in_specs=[
            pl.BlockSpec((d, tp), lambda i: (0, i)),
            pl.BlockSpec((cn * c, cn * cn), lambda i: (0, 0)),
        ],
        out_specs=pl.BlockSpec(
            (1, c, tp),
            lambda i, _t=tiles_per_batch: (i // _t, 0, i % _t)),
        compiler_params=pltpu.CompilerParams(
            dimension_semantics=("parallel",),
            vmem_limit_bytes=64 * 1024 * 1024,
        ),
    )(coords_t, code_r)

    return out.transpose(0, 2, 1)


# speedup vs baseline: 1.3566x; 1.0388x over previous
import functools

import jax
import jax.numpy as jnp
from jax.experimental import pallas as pl
from jax.experimental.pallas import tpu as pltpu

_CODE_NUM = 8
_D = 3
_IDX = 1


def _interp_kernel(coords_ref, code_ref, out_ref, *, cn, c, tp):
    scaled = (coords_ref[...] + 1.0) * ((cn - 1) / 2.0)
    grid_i = jax.lax.broadcasted_iota(jnp.int32, (cn, tp), 0).astype(jnp.float32)

    def hat(j):
        return jnp.maximum(0.0, 1.0 - jnp.abs(grid_i - scaled[j:j + 1, :]))

    h0 = hat(0)
    h1 = hat(1)
    h2 = hat(2)

    w_low = (h1[:, None, :] * h0[None, :, :]).reshape(cn * cn, tp)

    a = jnp.dot(code_ref[...], w_low,
                preferred_element_type=jnp.float32)

    acc = a[0:c, :] * h2[0:1, :]
    for i in range(1, cn):
        acc = acc + a[i * c:(i + 1) * c, :] * h2[i:i + 1, :]

    out_ref[0, :, :] = acc


@jax.jit
def kernel(coords, shape_code):
    b, p, d = coords.shape
    c = shape_code.shape[0]
    cn = _CODE_NUM
    nblk = cn ** d

    code = jax.lax.slice_in_dim(shape_code, _IDX * nblk, (_IDX + 1) * nblk, axis=1)
    code_r = (code.astype(jnp.float32)
              .reshape(c, cn, cn * cn)
              .transpose(1, 0, 2)
              .reshape(cn * c, cn * cn))

    npts = b * p
    tp = 32768
    if p % tp != 0:
        tp = 1024 if p % 1024 == 0 else p
    tiles_per_batch = p // tp
    kernel_fn = functools.partial(_interp_kernel, cn=cn, c=c, tp=tp)

    coords_t = coords.reshape(npts, d).T.astype(jnp.float32)

    out = pl.pallas_call(
        kernel_fn,
        out_shape=jax.ShapeDtypeStruct((b, c, p), jnp.float32),
        grid=(b * tiles_per_batch,),
        in_specs=[
            pl.BlockSpec((d, tp), lambda i: (0, i)),
            pl.BlockSpec((cn * c, cn * cn), lambda i: (0, 0)),
        ],
        out_specs=pl.BlockSpec(
            (1, c, tp),
            lambda i, _t=tiles_per_batch: (i // _t, 0, i % _t)),
        compiler_params=pltpu.CompilerParams(
            dimension_semantics=("parallel",),
            vmem_limit_bytes=64 * 1024 * 1024,
        ),
    )(coords_t, code_r)

    return out.transpose(0, 2, 1)


# speedup vs baseline: 1.4691x; 1.0829x over previous
import functools

import jax
import jax.numpy as jnp
from jax.experimental import pallas as pl
from jax.experimental.pallas import tpu as pltpu

_CODE_NUM = 8
_D = 3
_IDX = 1


def _interp_kernel(c0_ref, c1_ref, c2_ref, code_ref, out_ref, *, cn, c, tp):
    grid_i = jax.lax.broadcasted_iota(jnp.int32, (cn, tp), 0).astype(jnp.float32)

    def hat(cref):
        scaled = (cref[...] + 1.0) * ((cn - 1) / 2.0)
        return jnp.maximum(0.0, 1.0 - jnp.abs(grid_i - scaled))

    h0 = hat(c0_ref)
    h1 = hat(c1_ref)
    h2 = hat(c2_ref)

    w_low = (h1[:, None, :] * h0[None, :, :]).reshape(cn * cn, tp)

    a = jnp.dot(code_ref[...], w_low,
                preferred_element_type=jnp.float32)

    acc = a[0:c, :] * h2[0:1, :]
    for i in range(1, cn):
        acc = acc + a[i * c:(i + 1) * c, :] * h2[i:i + 1, :]

    out_ref[0, :, :] = acc


@jax.jit
def kernel(coords, shape_code):
    b, p, d = coords.shape
    c = shape_code.shape[0]
    cn = _CODE_NUM
    nblk = cn ** d

    code = jax.lax.slice_in_dim(shape_code, _IDX * nblk, (_IDX + 1) * nblk, axis=1)
    code_r = (code.astype(jnp.float32)
              .reshape(c, cn, cn * cn)
              .transpose(1, 0, 2)
              .reshape(cn * c, cn * cn))

    npts = b * p
    tp = 32768
    if p % tp != 0:
        tp = 1024 if p % 1024 == 0 else p
    tiles_per_batch = p // tp
    kernel_fn = functools.partial(_interp_kernel, cn=cn, c=c, tp=tp)

    planes = [coords[:, :, j].reshape(npts).astype(jnp.float32)
              for j in range(d)]

    out = pl.pallas_call(
        kernel_fn,
        out_shape=jax.ShapeDtypeStruct((b, c, p), jnp.float32),
        grid=(b * tiles_per_batch,),
        in_specs=[
            pl.BlockSpec((tp,), lambda i: (i,)),
            pl.BlockSpec((tp,), lambda i: (i,)),
            pl.BlockSpec((tp,), lambda i: (i,)),
            pl.BlockSpec((cn * c, cn * cn), lambda i: (0, 0)),
        ],
        out_specs=pl.BlockSpec(
            (1, c, tp),
            lambda i, _t=tiles_per_batch: (i // _t, 0, i % _t)),
        compiler_params=pltpu.CompilerParams(
            dimension_semantics=("parallel",),
            vmem_limit_bytes=64 * 1024 * 1024,
        ),
    )(*planes, code_r)

    return out.transpose(0, 2, 1)


# speedup vs baseline: 1.5034x; 1.0233x over previous
import functools

import jax
import jax.numpy as jnp
from jax.experimental import pallas as pl
from jax.experimental.pallas import tpu as pltpu

_CODE_NUM = 8
_D = 3
_IDX = 1


def _interp_kernel(c0_ref, c1_ref, c2_ref, code_ref, out_ref, *, cn, c, tp, sub):
    grid_i = jax.lax.broadcasted_iota(jnp.int32, (cn, sub), 0).astype(jnp.float32)

    def hat(cref, s):
        scaled = (cref[pl.ds(s * sub, sub)] + 1.0) * ((cn - 1) / 2.0)
        return jnp.maximum(0.0, 1.0 - jnp.abs(grid_i - scaled))

    for s in range(tp // sub):
        h0 = hat(c0_ref, s)
        h1 = hat(c1_ref, s)
        h2 = hat(c2_ref, s)

        w_low = (h1[:, None, :] * h0[None, :, :]).reshape(cn * cn, sub)

        a = jnp.dot(code_ref[...], w_low,
                    preferred_element_type=jnp.float32)

        acc = a[0:c, :] * h2[0:1, :]
        for i in range(1, cn):
            acc = acc + a[i * c:(i + 1) * c, :] * h2[i:i + 1, :]

        out_ref[0, :, pl.ds(s * sub, sub)] = acc


@jax.jit
def kernel(coords, shape_code):
    b, p, d = coords.shape
    c = shape_code.shape[0]
    cn = _CODE_NUM
    nblk = cn ** d

    npts = b * p
    tp = p
    sub = 8192
    if tp % sub != 0:
        sub = 1024 if tp % 1024 == 0 else tp

    code = jax.lax.slice_in_dim(shape_code, _IDX * nblk, (_IDX + 1) * nblk, axis=1)
    code_r = (code.astype(jnp.float32)
              .reshape(c, cn, cn * cn)
              .transpose(1, 0, 2)
              .reshape(cn * c, cn * cn))

    kernel_fn = functools.partial(_interp_kernel, cn=cn, c=c, tp=tp, sub=sub)

    planes = [coords[:, :, j].reshape(npts).astype(jnp.float32)
              for j in range(d)]

    out = pl.pallas_call(
        kernel_fn,
        out_shape=jax.ShapeDtypeStruct((b, c, p), jnp.float32),
        grid=(b,),
        in_specs=[
            pl.BlockSpec((tp,), lambda i: (i,)),
            pl.BlockSpec((tp,), lambda i: (i,)),
            pl.BlockSpec((tp,), lambda i: (i,)),
            pl.BlockSpec((cn * c, cn * cn), lambda i: (0, 0)),
        ],
        out_specs=pl.BlockSpec((1, c, tp), lambda i: (i, 0, 0)),
        compiler_params=pltpu.CompilerParams(
            dimension_semantics=("parallel",),
            vmem_limit_bytes=58 * 1024 * 1024,
        ),
    )(*planes, code_r)

    return out.transpose(0, 2, 1)
